# Initial kernel scaffold; baseline (speedup 1.0000x reference)
import jax
import jax.numpy as jnp
from jax.experimental import pallas as pl


def kernel(input_word_ids, emb_table, att_w, dense_w, dense_b):
    raise NotImplementedError("write your pallas kernel here")


# speedup vs baseline: 2.9190x; 2.9190x over previous
import functools

import jax
import jax.numpy as jnp
from jax import lax
from jax.experimental import pallas as pl
from jax.experimental.pallas import tpu as pltpu
from jax.experimental.pallas import tpu_sc as plsc

V = 260000
D = 128
B = 4096
L = 200
TOK = B * L

ROWS_BLK = 2048
VPAD = 260096
GRID1 = VPAD // ROWS_BLK

NC, NS = 2, 16
NW = NC * NS
CH = 128
ROWS_W = TOK // (NW * CH)
IDS_ROWS = TOK // CH


def _proj_body(x_ref, w_ref, s_ref, t_ref):
    x = x_ref[...]
    d = jnp.dot(x, w_ref[...], preferred_element_type=jnp.float32)
    s_ref[...] = d[:, 0]
    t_ref[...] = d[:, 1]


def _proj(table, w2):
    return pl.pallas_call(
        _proj_body,
        grid=(GRID1,),
        in_specs=[
            pl.BlockSpec((ROWS_BLK, D), lambda i: (i, 0)),
            pl.BlockSpec((D, 2), lambda i: (0, 0)),
        ],
        out_specs=[
            pl.BlockSpec((ROWS_BLK,), lambda i: (i,)),
            pl.BlockSpec((ROWS_BLK,), lambda i: (i,)),
        ],
        out_shape=[
            jax.ShapeDtypeStruct((VPAD,), jnp.float32),
            jax.ShapeDtypeStruct((VPAD,), jnp.float32),
        ],
    )(table, w2)


@functools.lru_cache(maxsize=1)
def _make_gather():
    mesh = plsc.VectorSubcoreMesh(core_axis_name="c", subcore_axis_name="s")

    @functools.partial(
        pl.kernel,
        mesh=mesh,
        out_type=[
            jax.ShapeDtypeStruct((IDS_ROWS, CH), jnp.float32),
            jax.ShapeDtypeStruct((IDS_ROWS, CH), jnp.float32),
        ],
        scratch_types=[
            pltpu.VMEM((ROWS_W, CH), jnp.int32),
            pltpu.VMEM((ROWS_W, CH), jnp.float32),
            pltpu.VMEM((ROWS_W, CH), jnp.float32),
            pltpu.SemaphoreType.DMA,
            pltpu.SemaphoreType.DMA,
        ],
    )
    def _gather_st(s_hbm, t_hbm, ids_hbm, sg_hbm, tg_hbm,
                   idx_v, sg_v, tg_v, sem_s, sem_t):
        wid = lax.axis_index("s") * NC + lax.axis_index("c")
        base = wid * ROWS_W
        pltpu.sync_copy(ids_hbm.at[pl.ds(base, ROWS_W)], idx_v)

        unroll = 8

        def chunk(k, carry):
            cps = []
            for i in range(unroll):
                j = k * unroll + i
                cps.append(pltpu.async_copy(s_hbm.at[idx_v.at[j]], sg_v.at[j], sem_s))
                cps.append(pltpu.async_copy(t_hbm.at[idx_v.at[j]], tg_v.at[j], sem_t))
            for c in cps:
                c.wait()
            return carry

        lax.fori_loop(0, ROWS_W // unroll, chunk, 0)
        pltpu.sync_copy(sg_v, sg_hbm.at[pl.ds(base, ROWS_W)])
        pltpu.sync_copy(tg_v, tg_hbm.at[pl.ds(base, ROWS_W)])

    return _gather_st


POOL_BLK = 512


def _pool_body(s_ref, t_ref, o_ref):
    s = s_ref[...]
    t = t_ref[...]
    m = jnp.max(s, axis=1, keepdims=True)
    e = jnp.exp(s - m)
    num = jnp.sum(e * t, axis=1, keepdims=True)
    den = jnp.sum(e, axis=1, keepdims=True)
    o_ref[...] = num / den


def _pool(s_tok, t_tok):
    return pl.pallas_call(
        _pool_body,
        grid=(B // POOL_BLK,),
        in_specs=[
            pl.BlockSpec((POOL_BLK, L), lambda i: (i, 0)),
            pl.BlockSpec((POOL_BLK, L), lambda i: (i, 0)),
        ],
        out_specs=pl.BlockSpec((POOL_BLK, 1), lambda i: (i, 0)),
        out_shape=jax.ShapeDtypeStruct((B, 1), jnp.float32),
    )(s_tok, t_tok)


def kernel(input_word_ids, emb_table, att_w, dense_w, dense_b):
    w2 = jnp.concatenate([att_w, dense_w], axis=1)
    s_words, t_words = _proj(emb_table, w2)
    ids2d = input_word_ids.astype(jnp.int32).reshape(IDS_ROWS, CH)
    sg, tg = _make_gather()(s_words, t_words, ids2d)
    s_tok = sg.reshape(B, L)
    t_tok = tg.reshape(B, L)
    out = _pool(s_tok, t_tok)
    return out + dense_b[None, :]


# speedup vs baseline: 3.8196x; 1.3085x over previous
import functools

import jax
import jax.numpy as jnp
from jax import lax
from jax.experimental import pallas as pl
from jax.experimental.pallas import tpu as pltpu
from jax.experimental.pallas import tpu_sc as plsc

V = 260000
D = 128
B = 4096
L = 200
TOK = B * L

ROWS_BLK = 2048
VPAD = 260096
GRID1 = VPAD // ROWS_BLK

NC, NS = 2, 16
NW = NC * NS
CH = 128
ROWS_W = TOK // (NW * CH)
IDS_ROWS = TOK // CH


def _proj_body(x_ref, w_ref, s_ref, t_ref):
    x = x_ref[...]
    d = lax.dot_general(w_ref[...], x, (((1,), (1,)), ((), ())),
                        preferred_element_type=jnp.float32)
    s_ref[...] = d[0]
    t_ref[...] = d[1]


def _proj(table, w2t):
    return pl.pallas_call(
        _proj_body,
        grid=(GRID1,),
        in_specs=[
            pl.BlockSpec((ROWS_BLK, D), lambda i: (i, 0)),
            pl.BlockSpec((2, D), lambda i: (0, 0)),
        ],
        out_specs=[
            pl.BlockSpec((ROWS_BLK,), lambda i: (i,)),
            pl.BlockSpec((ROWS_BLK,), lambda i: (i,)),
        ],
        out_shape=[
            jax.ShapeDtypeStruct((VPAD,), jnp.float32),
            jax.ShapeDtypeStruct((VPAD,), jnp.float32),
        ],
    )(table, w2t)


@functools.lru_cache(maxsize=1)
def _make_gather():
    mesh = plsc.VectorSubcoreMesh(core_axis_name="c", subcore_axis_name="s")

    @functools.partial(
        pl.kernel,
        mesh=mesh,
        out_type=[
            jax.ShapeDtypeStruct((IDS_ROWS, CH), jnp.float32),
            jax.ShapeDtypeStruct((IDS_ROWS, CH), jnp.float32),
        ],
        scratch_types=[
            pltpu.VMEM((ROWS_W, CH), jnp.int32),
            pltpu.VMEM((ROWS_W, CH), jnp.float32),
            pltpu.VMEM((ROWS_W, CH), jnp.float32),
            pltpu.SemaphoreType.DMA,
            pltpu.SemaphoreType.DMA,
        ],
    )
    def _gather_st(s_hbm, t_hbm, ids_hbm, sg_hbm, tg_hbm,
                   idx_v, sg_v, tg_v, sem_s, sem_t):
        wid = lax.axis_index("s") * NC + lax.axis_index("c")
        base = wid * ROWS_W
        pltpu.sync_copy(ids_hbm.at[pl.ds(base, ROWS_W)], idx_v)

        unroll = 8

        def chunk(k, carry):
            cps = []
            for i in range(unroll):
                j = k * unroll + i
                cps.append(pltpu.async_copy(s_hbm.at[idx_v.at[j]], sg_v.at[j], sem_s))
                cps.append(pltpu.async_copy(t_hbm.at[idx_v.at[j]], tg_v.at[j], sem_t))
            for c in cps:
                c.wait()
            return carry

        lax.fori_loop(0, ROWS_W // unroll, chunk, 0)
        pltpu.sync_copy(sg_v, sg_hbm.at[pl.ds(base, ROWS_W)])
        pltpu.sync_copy(tg_v, tg_hbm.at[pl.ds(base, ROWS_W)])

    return _gather_st


POOL_BLK = 512


def _pool_body(s_ref, t_ref, o_ref):
    s = s_ref[...]
    t = t_ref[...]
    m = jnp.max(s, axis=1, keepdims=True)
    e = jnp.exp(s - m)
    num = jnp.sum(e * t, axis=1, keepdims=True)
    den = jnp.sum(e, axis=1, keepdims=True)
    o_ref[...] = num / den


def _pool(s_tok, t_tok):
    return pl.pallas_call(
        _pool_body,
        grid=(B // POOL_BLK,),
        in_specs=[
            pl.BlockSpec((POOL_BLK, L), lambda i: (i, 0)),
            pl.BlockSpec((POOL_BLK, L), lambda i: (i, 0)),
        ],
        out_specs=pl.BlockSpec((POOL_BLK, 1), lambda i: (i, 0)),
        out_shape=jax.ShapeDtypeStruct((B, 1), jnp.float32),
    )(s_tok, t_tok)


def kernel(input_word_ids, emb_table, att_w, dense_w, dense_b):
    w2t = jnp.concatenate([att_w, dense_w], axis=1).T
    s_words, t_words = _proj(emb_table, w2t)
    ids2d = input_word_ids.astype(jnp.int32).reshape(IDS_ROWS, CH)
    sg, tg = _make_gather()(s_words, t_words, ids2d)
    s_tok = sg.reshape(B, L)
    t_tok = tg.reshape(B, L)
    out = _pool(s_tok, t_tok)
    return out + dense_b[None, :]


# speedup vs baseline: 4.2791x; 1.1203x over previous
import functools

import jax
import jax.numpy as jnp
from jax import lax
from jax.experimental import pallas as pl
from jax.experimental.pallas import tpu as pltpu
from jax.experimental.pallas import tpu_sc as plsc

V = 260000
D = 128
B = 4096
L = 200
TOK = B * L

ROWS_BLK = 2048
VPAD = 260096
GRID1 = VPAD // ROWS_BLK

NC, NS = 2, 16
NW = NC * NS
CH = 128
ROWS_W = TOK // (NW * CH)
IDS_ROWS = TOK // CH


def _proj_body(x_ref, w_ref, s_ref, t_ref):
    x = x_ref[...]
    d = lax.dot_general(w_ref[...], x, (((1,), (1,)), ((), ())),
                        preferred_element_type=jnp.float32)
    s_ref[...] = d[0]
    t_ref[...] = d[1]


def _proj(table, w2t):
    return pl.pallas_call(
        _proj_body,
        grid=(GRID1,),
        in_specs=[
            pl.BlockSpec((ROWS_BLK, D), lambda i: (i, 0)),
            pl.BlockSpec((2, D), lambda i: (0, 0)),
        ],
        out_specs=[
            pl.BlockSpec((ROWS_BLK,), lambda i: (i,)),
            pl.BlockSpec((ROWS_BLK,), lambda i: (i,)),
        ],
        out_shape=[
            jax.ShapeDtypeStruct((VPAD,), jnp.float32),
            jax.ShapeDtypeStruct((VPAD,), jnp.float32),
        ],
    )(table, w2t)


ROWS_B = B // NW
GROUPS = ROWS_B // 16


@functools.lru_cache(maxsize=1)
def _make_gather_pool():
    mesh = plsc.VectorSubcoreMesh(core_axis_name="c", subcore_axis_name="s")

    @functools.partial(
        pl.kernel,
        mesh=mesh,
        out_type=jax.ShapeDtypeStruct((B,), jnp.float32),
        scratch_types=[
            pltpu.VMEM((L, ROWS_B), jnp.int32),
            pltpu.VMEM((L * ROWS_B,), jnp.float32),
            pltpu.VMEM((L * ROWS_B,), jnp.float32),
            pltpu.VMEM((ROWS_B,), jnp.float32),
            pltpu.SemaphoreType.DMA,
            pltpu.SemaphoreType.DMA,
        ],
    )
    def _gather_pool(s_hbm, t_hbm, ids_hbm, out_hbm,
                     idx_v, sg_v, tg_v, res_v, sem_s, sem_t):
        wid = lax.axis_index("s") * NC + lax.axis_index("c")
        pltpu.sync_copy(ids_hbm.at[pl.ds(wid * L, L)], idx_v)

        unroll = 8

        def chunk(k, carry):
            cps = []
            for i in range(unroll):
                j = k * unroll + i
                cps.append(pltpu.async_copy(
                    s_hbm.at[idx_v.at[j]], sg_v.at[pl.ds(j * ROWS_B, ROWS_B)], sem_s))
                cps.append(pltpu.async_copy(
                    t_hbm.at[idx_v.at[j]], tg_v.at[pl.ds(j * ROWS_B, ROWS_B)], sem_t))
            for c in cps:
                c.wait()
            return carry

        lax.fori_loop(0, L // unroll, chunk, 0)

        for g in range(GROUPS):
            off = g * 16

            def pass_max(p, m):
                s = sg_v[pl.ds(p * ROWS_B + off, 16)]
                return jnp.maximum(m, s)

            m = lax.fori_loop(0, L, pass_max,
                              jnp.full((16,), -jnp.inf, jnp.float32))

            def pass_acc(p, carry):
                num, den = carry
                s = sg_v[pl.ds(p * ROWS_B + off, 16)]
                t = tg_v[pl.ds(p * ROWS_B + off, 16)]
                e = jnp.exp(s - m)
                return num + e * t, den + e

            num, den = lax.fori_loop(
                0, L, pass_acc,
                (jnp.zeros((16,), jnp.float32), jnp.zeros((16,), jnp.float32)))
            res_v[pl.ds(off, 16)] = num / den

        pltpu.sync_copy(res_v, out_hbm.at[pl.ds(wid * ROWS_B, ROWS_B)])

    return _gather_pool


def kernel(input_word_ids, emb_table, att_w, dense_w, dense_b):
    w2t = jnp.concatenate([att_w, dense_w], axis=1).T
    s_words, t_words = _proj(emb_table, w2t)
    ids_pm = (input_word_ids.astype(jnp.int32)
              .reshape(NW, ROWS_B, L)
              .transpose(0, 2, 1)
              .reshape(NW * L, ROWS_B))
    out = _make_gather_pool()(s_words, t_words, ids_pm)
    return out[:, None] + dense_b[None, :]


# speedup vs baseline: 5.6392x; 1.3179x over previous
import functools

import jax
import jax.numpy as jnp
from jax import lax
from jax.experimental import pallas as pl
from jax.experimental.pallas import tpu as pltpu
from jax.experimental.pallas import tpu_sc as plsc

V = 260000
D = 128
B = 4096
L = 200
TOK = B * L

ROWS_BLK = 8192
VPAD = 262144
GRID1 = VPAD // ROWS_BLK

NC, NS = 2, 16
NW = NC * NS
CH = 128
ROWS_W = TOK // (NW * CH)
IDS_ROWS = TOK // CH


def _proj_body(x_ref, w_ref, s_ref, t_ref):
    x = x_ref[...]
    d = lax.dot_general(w_ref[...], x, (((1,), (1,)), ((), ())),
                        preferred_element_type=jnp.float32)
    s_ref[...] = d[0]
    t_ref[...] = d[1]


def _proj(table, w2t):
    return pl.pallas_call(
        _proj_body,
        grid=(GRID1,),
        in_specs=[
            pl.BlockSpec((ROWS_BLK, D), lambda i: (i, 0)),
            pl.BlockSpec((2, D), lambda i: (0, 0)),
        ],
        out_specs=[
            pl.BlockSpec((ROWS_BLK,), lambda i: (i,)),
            pl.BlockSpec((ROWS_BLK,), lambda i: (i,)),
        ],
        out_shape=[
            jax.ShapeDtypeStruct((VPAD,), jnp.float32),
            jax.ShapeDtypeStruct((VPAD,), jnp.float32),
        ],
    )(table, w2t)


ROWS_B = B // NW
GROUPS = ROWS_B // 16


@functools.lru_cache(maxsize=1)
def _make_gather_pool():
    mesh = plsc.VectorSubcoreMesh(core_axis_name="c", subcore_axis_name="s")

    @functools.partial(
        pl.kernel,
        mesh=mesh,
        out_type=jax.ShapeDtypeStruct((B,), jnp.float32),
        scratch_types=[
            pltpu.VMEM((L, ROWS_B), jnp.int32),
            pltpu.VMEM((L * ROWS_B,), jnp.float32),
            pltpu.VMEM((L * ROWS_B,), jnp.float32),
            pltpu.VMEM((ROWS_B,), jnp.float32),
            pltpu.SemaphoreType.DMA,
            pltpu.SemaphoreType.DMA,
        ],
    )
    def _gather_pool(s_hbm, t_hbm, ids_hbm, out_hbm,
                     idx_v, sg_v, tg_v, res_v, sem_s, sem_t):
        wid = lax.axis_index("s") * NC + lax.axis_index("c")
        pltpu.sync_copy(ids_hbm.at[pl.ds(wid * L, L)], idx_v)

        unroll = 8

        def chunk(k, carry):
            cps = []
            for i in range(unroll):
                j = k * unroll + i
                cps.append(pltpu.async_copy(
                    s_hbm.at[idx_v.at[j]], sg_v.at[pl.ds(j * ROWS_B, ROWS_B)], sem_s))
                cps.append(pltpu.async_copy(
                    t_hbm.at[idx_v.at[j]], tg_v.at[pl.ds(j * ROWS_B, ROWS_B)], sem_t))
            for c in cps:
                c.wait()
            return carry

        lax.fori_loop(0, L // unroll, chunk, 0)

        for g in range(GROUPS):
            off = g * 16

            def pass_max(p, m):
                s = sg_v[pl.ds(p * ROWS_B + off, 16)]
                return jnp.maximum(m, s)

            m = lax.fori_loop(0, L, pass_max,
                              jnp.full((16,), -jnp.inf, jnp.float32))

            def pass_acc(p, carry):
                num, den = carry
                s = sg_v[pl.ds(p * ROWS_B + off, 16)]
                t = tg_v[pl.ds(p * ROWS_B + off, 16)]
                e = jnp.exp(s - m)
                return num + e * t, den + e

            num, den = lax.fori_loop(
                0, L, pass_acc,
                (jnp.zeros((16,), jnp.float32), jnp.zeros((16,), jnp.float32)))
            res_v[pl.ds(off, 16)] = num / den

        pltpu.sync_copy(res_v, out_hbm.at[pl.ds(wid * ROWS_B, ROWS_B)])

    return _gather_pool


def kernel(input_word_ids, emb_table, att_w, dense_w, dense_b):
    w2t = jnp.concatenate([att_w, dense_w], axis=1).T
    s_words, t_words = _proj(emb_table, w2t)
    ids_pm = (input_word_ids.astype(jnp.int32)
              .reshape(NW, ROWS_B, L)
              .transpose(0, 2, 1)
              .reshape(NW * L, ROWS_B))
    out = _make_gather_pool()(s_words, t_words, ids_pm)
    return out[:, None] + dense_b[None, :]


# speedup vs baseline: 8.0338x; 1.4246x over previous
import functools

import jax
import jax.numpy as jnp
from jax import lax
from jax.experimental import pallas as pl
from jax.experimental.pallas import tpu as pltpu
from jax.experimental.pallas import tpu_sc as plsc

V = 260000
D = 128
B = 4096
L = 200
TOK = B * L

ROWS_BLK = 8192
VPAD = 262144
GRID1 = VPAD // ROWS_BLK

NC, NS = 2, 16
NW = NC * NS
CH = 128
ROWS_W = TOK // (NW * CH)
IDS_ROWS = TOK // CH


def _proj_body(x_ref, w_ref, s_ref, t_ref):
    x = x_ref[...]
    d = lax.dot_general(w_ref[...], x, (((1,), (1,)), ((), ())),
                        preferred_element_type=jnp.float32)
    s_ref[...] = d[0]
    t_ref[...] = d[1]


def _proj(table, w2t):
    return pl.pallas_call(
        _proj_body,
        grid=(GRID1,),
        in_specs=[
            pl.BlockSpec((ROWS_BLK, D), lambda i: (i, 0)),
            pl.BlockSpec((2, D), lambda i: (0, 0)),
        ],
        out_specs=[
            pl.BlockSpec((ROWS_BLK,), lambda i: (i,)),
            pl.BlockSpec((ROWS_BLK,), lambda i: (i,)),
        ],
        out_shape=[
            jax.ShapeDtypeStruct((VPAD,), jnp.float32),
            jax.ShapeDtypeStruct((VPAD,), jnp.float32),
        ],
    )(table, w2t)


ROWS_B = B // NW
GROUPS = ROWS_B // 16


@functools.lru_cache(maxsize=1)
def _make_gather_pool():
    mesh = plsc.VectorSubcoreMesh(core_axis_name="c", subcore_axis_name="s")

    @functools.partial(
        pl.kernel,
        mesh=mesh,
        out_type=jax.ShapeDtypeStruct((B,), jnp.float32),
        scratch_types=[
            pltpu.VMEM((L, ROWS_B), jnp.int32),
            pltpu.VMEM((L * ROWS_B,), jnp.float32),
            pltpu.VMEM((L * ROWS_B,), jnp.float32),
            pltpu.VMEM((ROWS_B,), jnp.float32),
            pltpu.VMEM_SHARED((VPAD,), jnp.float32),
            pltpu.VMEM_SHARED((VPAD,), jnp.float32),
            pltpu.SemaphoreType.DMA,
            pltpu.SemaphoreType.DMA,
        ],
    )
    def _gather_pool(s_hbm, t_hbm, ids_hbm, out_hbm,
                     idx_v, sg_v, tg_v, res_v, s_spm, t_spm, sem_s, sem_t):
        wid = lax.axis_index("s") * NC + lax.axis_index("c")
        pltpu.sync_copy(ids_hbm.at[pl.ds(wid * L, L)], idx_v)

        sid = lax.axis_index("s")
        slc = VPAD // NS
        pltpu.sync_copy(s_hbm.at[pl.ds(sid * slc, slc)],
                        s_spm.at[pl.ds(sid * slc, slc)])
        pltpu.sync_copy(t_hbm.at[pl.ds(sid * slc, slc)],
                        t_spm.at[pl.ds(sid * slc, slc)])
        plsc.subcore_barrier()

        unroll = 8

        def chunk(k, carry):
            cps = []
            for i in range(unroll):
                j = k * unroll + i
                cps.append(pltpu.async_copy(
                    s_spm.at[idx_v.at[j]], sg_v.at[pl.ds(j * ROWS_B, ROWS_B)], sem_s))
                cps.append(pltpu.async_copy(
                    t_spm.at[idx_v.at[j]], tg_v.at[pl.ds(j * ROWS_B, ROWS_B)], sem_t))
            for c in cps:
                c.wait()
            return carry

        lax.fori_loop(0, L // unroll, chunk, 0)

        for g in range(GROUPS):
            off = g * 16

            def pass_max(p, m):
                s = sg_v[pl.ds(p * ROWS_B + off, 16)]
                return jnp.maximum(m, s)

            m = lax.fori_loop(0, L, pass_max,
                              jnp.full((16,), -jnp.inf, jnp.float32))

            def pass_acc(p, carry):
                num, den = carry
                s = sg_v[pl.ds(p * ROWS_B + off, 16)]
                t = tg_v[pl.ds(p * ROWS_B + off, 16)]
                e = jnp.exp(s - m)
                return num + e * t, den + e

            num, den = lax.fori_loop(
                0, L, pass_acc,
                (jnp.zeros((16,), jnp.float32), jnp.zeros((16,), jnp.float32)))
            res_v[pl.ds(off, 16)] = num / den

        pltpu.sync_copy(res_v, out_hbm.at[pl.ds(wid * ROWS_B, ROWS_B)])

    return _gather_pool


def kernel(input_word_ids, emb_table, att_w, dense_w, dense_b):
    w2t = jnp.concatenate([att_w, dense_w], axis=1).T
    s_words, t_words = _proj(emb_table, w2t)
    ids_pm = (input_word_ids.astype(jnp.int32)
              .reshape(NW, ROWS_B, L)
              .transpose(0, 2, 1)
              .reshape(NW * L, ROWS_B))
    out = _make_gather_pool()(s_words, t_words, ids_pm)
    return out[:, None] + dense_b[None, :]


# speedup vs baseline: 9.5520x; 1.1890x over previous
import functools

import jax
import jax.numpy as jnp
from jax import lax
from jax.experimental import pallas as pl
from jax.experimental.pallas import tpu as pltpu
from jax.experimental.pallas import tpu_sc as plsc

V = 260000
D = 128
B = 4096
L = 200
TOK = B * L

ROWS_BLK = 16384
VPAD = 262144
GRID1 = VPAD // ROWS_BLK

NC, NS = 2, 16
NW = NC * NS
CH = 128
ROWS_W = TOK // (NW * CH)
IDS_ROWS = TOK // CH


def _proj_body(x_ref, w_ref, s_ref, t_ref):
    x = x_ref[...]
    d = lax.dot_general(w_ref[...], x, (((1,), (1,)), ((), ())),
                        preferred_element_type=jnp.float32)
    s_ref[...] = d[0]
    t_ref[...] = d[1]


def _proj(table, w2t):
    return pl.pallas_call(
        _proj_body,
        grid=(GRID1,),
        in_specs=[
            pl.BlockSpec((ROWS_BLK, D), lambda i: (i, 0)),
            pl.BlockSpec((2, D), lambda i: (0, 0)),
        ],
        out_specs=[
            pl.BlockSpec((ROWS_BLK,), lambda i: (i,)),
            pl.BlockSpec((ROWS_BLK,), lambda i: (i,)),
        ],
        out_shape=[
            jax.ShapeDtypeStruct((VPAD,), jnp.float32),
            jax.ShapeDtypeStruct((VPAD,), jnp.float32),
        ],
    )(table, w2t)


ROWS_B = B // NW
GROUPS = ROWS_B // 16


@functools.lru_cache(maxsize=1)
def _make_gather_pool():
    mesh = plsc.VectorSubcoreMesh(core_axis_name="c", subcore_axis_name="s")

    @functools.partial(
        pl.kernel,
        mesh=mesh,
        out_type=jax.ShapeDtypeStruct((B,), jnp.float32),
        scratch_types=[
            pltpu.VMEM((L, ROWS_B), jnp.int32),
            pltpu.VMEM((L * ROWS_B,), jnp.float32),
            pltpu.VMEM((L * ROWS_B,), jnp.float32),
            pltpu.VMEM((ROWS_B,), jnp.float32),
            pltpu.VMEM_SHARED((VPAD,), jnp.float32),
            pltpu.VMEM_SHARED((VPAD,), jnp.float32),
            pltpu.SemaphoreType.DMA,
            pltpu.SemaphoreType.DMA,
        ],
    )
    def _gather_pool(s_hbm, t_hbm, ids_hbm, out_hbm,
                     idx_v, sg_v, tg_v, res_v, s_spm, t_spm, sem_s, sem_t):
        wid = lax.axis_index("s") * NC + lax.axis_index("c")
        pltpu.sync_copy(ids_hbm.at[pl.ds(wid * L, L)], idx_v)

        sid = lax.axis_index("s")
        slc = VPAD // NS
        pltpu.sync_copy(s_hbm.at[pl.ds(sid * slc, slc)],
                        s_spm.at[pl.ds(sid * slc, slc)])
        pltpu.sync_copy(t_hbm.at[pl.ds(sid * slc, slc)],
                        t_spm.at[pl.ds(sid * slc, slc)])
        plsc.subcore_barrier()

        unroll = 8

        def chunk(k, carry):
            ms, nums, dens = carry
            cps = []
            for i in range(unroll):
                j = k * unroll + i
                cps.append(pltpu.async_copy(
                    s_spm.at[idx_v.at[j]], sg_v.at[pl.ds(j * ROWS_B, ROWS_B)], sem_s))
                cps.append(pltpu.async_copy(
                    t_spm.at[idx_v.at[j]], tg_v.at[pl.ds(j * ROWS_B, ROWS_B)], sem_t))
            for c in cps:
                c.wait()
            ms, nums, dens = list(ms), list(nums), list(dens)
            for i in range(unroll):
                j = k * unroll + i
                for g in range(GROUPS):
                    off = j * ROWS_B + g * 16
                    s = sg_v[pl.ds(off, 16)]
                    t = tg_v[pl.ds(off, 16)]
                    m_new = jnp.maximum(ms[g], s)
                    scale = jnp.exp(ms[g] - m_new)
                    e = jnp.exp(s - m_new)
                    nums[g] = nums[g] * scale + e * t
                    dens[g] = dens[g] * scale + e
                    ms[g] = m_new
            return tuple(ms), tuple(nums), tuple(dens)

        neg_inf = jnp.full((16,), -jnp.inf, jnp.float32)
        zero = jnp.zeros((16,), jnp.float32)
        ms, nums, dens = lax.fori_loop(
            0, L // unroll, chunk,
            (tuple(neg_inf for _ in range(GROUPS)),
             tuple(zero for _ in range(GROUPS)),
             tuple(zero for _ in range(GROUPS))))
        for g in range(GROUPS):
            res_v[pl.ds(g * 16, 16)] = nums[g] / dens[g]

        pltpu.sync_copy(res_v, out_hbm.at[pl.ds(wid * ROWS_B, ROWS_B)])

    return _gather_pool


def kernel(input_word_ids, emb_table, att_w, dense_w, dense_b):
    w2t = jnp.concatenate([att_w, dense_w], axis=1).T
    s_words, t_words = _proj(emb_table, w2t)
    ids_pm = (input_word_ids.astype(jnp.int32)
              .reshape(NW, ROWS_B, L)
              .transpose(0, 2, 1)
              .reshape(NW * L, ROWS_B))
    out = _make_gather_pool()(s_words, t_words, ids_pm)
    return out[:, None] + dense_b[None, :]
